# Initial kernel scaffold; baseline (speedup 1.0000x reference)
import jax
import jax.numpy as jnp
from jax.experimental import pallas as pl


def kernel(input, weight):
    raise NotImplementedError("write your pallas kernel here")


# speedup vs baseline: 1.7950x; 1.7950x over previous
import functools

import jax
import jax.numpy as jnp
from jax import lax
from jax.experimental import pallas as pl
from jax.experimental.pallas import tpu as pltpu
from jax.experimental.pallas import tpu_sc as plsc

NC = 2
NS = 16
NW = NC * NS

D = 64
CHUNK = 512


def _gather_body(table_hbm, idx_hbm, out_hbm, idx_v, rows_v, sem, *, n_per_w):
    wid = lax.axis_index("s") * NC + lax.axis_index("c")
    base = wid * n_per_w
    nchunks = n_per_w // CHUNK

    def step(i, _):
        off = base + i * CHUNK
        pltpu.sync_copy(idx_hbm.at[pl.ds(off, CHUNK)], idx_v)
        pltpu.async_copy(table_hbm.at[idx_v], rows_v, sem).wait()
        pltpu.sync_copy(rows_v, out_hbm.at[pl.ds(off, CHUNK)])
        return 0

    lax.fori_loop(0, nchunks, step, 0)


def kernel(input, weight):
    B = input.shape[0] * input.shape[1]
    n_per_w = B // NW
    idx_flat = input.reshape(B)

    mesh = plsc.VectorSubcoreMesh(core_axis_name="c", subcore_axis_name="s")
    k = functools.partial(
        pl.kernel,
        out_type=jax.ShapeDtypeStruct((B, D), jnp.float32),
        mesh=mesh,
        scratch_types=[
            pltpu.VMEM((CHUNK,), jnp.int32),
            pltpu.VMEM((CHUNK, D), jnp.float32),
            pltpu.SemaphoreType.DMA,
        ],
        compiler_params=pltpu.CompilerParams(use_tc_tiling_on_sc=False),
    )(functools.partial(_gather_body, n_per_w=n_per_w))
    out = k(weight, idx_flat)
    return out.reshape(input.shape[0], input.shape[1], D)


# speedup vs baseline: 1.8648x; 1.0389x over previous
import functools

import jax
import jax.numpy as jnp
from jax import lax
from jax.experimental import pallas as pl
from jax.experimental.pallas import tpu as pltpu
from jax.experimental.pallas import tpu_sc as plsc

NC = 2
NS = 16
NW = NC * NS

D = 64
CHUNK = 512
NBUF = 2


def _gather_body(table_hbm, idx_hbm, out_hbm, idx_v, rows_v, *sems, n_per_w):
    gsems, ssems = sems[:NBUF], sems[NBUF:]
    wid = lax.axis_index("s") * NC + lax.axis_index("c")
    base = wid * n_per_w
    nchunks = n_per_w // CHUNK
    ngroups = nchunks // NBUF

    pltpu.sync_copy(idx_hbm.at[pl.ds(base, n_per_w)], idx_v)

    def gather(g, b):
        pltpu.async_copy(
            table_hbm.at[idx_v.at[pl.ds(g * CHUNK, CHUNK)]], rows_v.at[b], gsems[b])

    def gather_wait(g, b):
        pltpu.make_async_copy(
            table_hbm.at[idx_v.at[pl.ds(g * CHUNK, CHUNK)]], rows_v.at[b], gsems[b]).wait()

    def store(g, b):
        pltpu.async_copy(
            rows_v.at[b], out_hbm.at[pl.ds(base + g * CHUNK, CHUNK)], ssems[b])

    def store_wait(g, b):
        pltpu.make_async_copy(
            rows_v.at[b], out_hbm.at[pl.ds(base + g * CHUNK, CHUNK)], ssems[b]).wait()

    for b in range(NBUF):
        gather(b, b)

    def group(j, _):
        for b in range(NBUF):
            g_prev = (j - 1) * NBUF + b
            gather_wait(g_prev, b)
            store(g_prev, b)
        for b in range(NBUF):
            g = j * NBUF + b
            store_wait((j - 1) * NBUF + b, b)
            gather(g, b)
        return 0

    lax.fori_loop(1, ngroups, group, 0)

    for b in range(NBUF):
        g_last = (ngroups - 1) * NBUF + b
        gather_wait(g_last, b)
        store(g_last, b)
    for b in range(NBUF):
        store_wait((ngroups - 1) * NBUF + b, b)


def kernel(input, weight):
    B = input.shape[0] * input.shape[1]
    n_per_w = B // NW
    idx_flat = input.reshape(B)

    mesh = plsc.VectorSubcoreMesh(core_axis_name="c", subcore_axis_name="s")
    k = functools.partial(
        pl.kernel,
        out_type=jax.ShapeDtypeStruct((B, D), jnp.float32),
        mesh=mesh,
        scratch_types=[
            pltpu.VMEM((n_per_w,), jnp.int32),
            pltpu.VMEM((NBUF, CHUNK, D), jnp.float32),
        ] + [pltpu.SemaphoreType.DMA] * (2 * NBUF),
        compiler_params=pltpu.CompilerParams(use_tc_tiling_on_sc=False),
    )(functools.partial(_gather_body, n_per_w=n_per_w))
    out = k(weight, idx_flat)
    return out.reshape(input.shape[0], input.shape[1], D)


# speedup vs baseline: 1.8754x; 1.0057x over previous
import functools

import jax
import jax.numpy as jnp
from jax import lax
from jax.experimental import pallas as pl
from jax.experimental.pallas import tpu as pltpu
from jax.experimental.pallas import tpu_sc as plsc

NC = 2
NS = 16
NW = NC * NS

D = 64
CHUNK = 128
NBUF = 8


def _gather_body(table_hbm, idx_hbm, out_hbm, idx_v, rows_v, *gsems, n_per_w):
    wid = lax.axis_index("s") * NC + lax.axis_index("c")
    base = wid * n_per_w
    nchunks = n_per_w // CHUNK
    ngroups = nchunks // NBUF

    pltpu.sync_copy(idx_hbm.at[pl.ds(base, n_per_w)], idx_v)

    def gather(g, b):
        pltpu.async_copy(
            table_hbm.at[idx_v.at[pl.ds(g * CHUNK, CHUNK)]], rows_v.at[b], gsems[b])

    def gather_wait(g, b):
        pltpu.make_async_copy(
            table_hbm.at[idx_v.at[pl.ds(g * CHUNK, CHUNK)]], rows_v.at[b], gsems[b]).wait()

    def store(g, b):
        pltpu.sync_copy(rows_v.at[b], out_hbm.at[pl.ds(base + g * CHUNK, CHUNK)])

    for b in range(NBUF):
        gather(b, b)

    def group(j, _):
        for b in range(NBUF):
            g = j * NBUF + b
            gather_wait(g, b)
            store(g, b)
            gather(g + NBUF, b)
        return 0

    lax.fori_loop(0, ngroups - 1, group, 0)

    for b in range(NBUF):
        g = (ngroups - 1) * NBUF + b
        gather_wait(g, b)
        store(g, b)


def kernel(input, weight):
    B = input.shape[0] * input.shape[1]
    n_per_w = B // NW
    idx_flat = input.reshape(B)

    mesh = plsc.VectorSubcoreMesh(core_axis_name="c", subcore_axis_name="s")
    k = functools.partial(
        pl.kernel,
        out_type=jax.ShapeDtypeStruct((B, D), jnp.float32),
        mesh=mesh,
        scratch_types=[
            pltpu.VMEM((n_per_w,), jnp.int32),
            pltpu.VMEM((NBUF, CHUNK, D), jnp.float32),
        ] + [pltpu.SemaphoreType.DMA] * NBUF,
        compiler_params=pltpu.CompilerParams(use_tc_tiling_on_sc=False),
    )(functools.partial(_gather_body, n_per_w=n_per_w))
    out = k(weight, idx_flat)
    return out.reshape(input.shape[0], input.shape[1], D)
